# baseline (speedup 1.0000x reference)
import functools

import jax
import jax.numpy as jnp
from jax import lax
from jax.experimental import pallas as pl
from jax.experimental.pallas import tpu as pltpu
from jax.experimental.pallas import tpu_sc as plsc

VOCAB = 100000
DIM = 64
MAXLEN = 200
BATCH = 1024

NC = 2
NS = 16
NW = NC * NS
ROWS_PER_W = BATCH // NW

IDX_SPLIT = 2
IDX_CHUNK = MAXLEN // IDX_SPLIT

NBUF = 4


def _make_kernel():
  mesh = plsc.VectorSubcoreMesh(core_axis_name="c", subcore_axis_name="s")

  @functools.partial(
      pl.kernel,
      out_type=jax.ShapeDtypeStruct((BATCH, MAXLEN, DIM), jnp.float32),
      mesh=mesh,
      scratch_types=[
          pltpu.VMEM((ROWS_PER_W, IDX_SPLIT, IDX_CHUNK), jnp.int32),
      ]
      + [pltpu.VMEM((MAXLEN, DIM), jnp.float32) for _ in range(NBUF)]
      + [pltpu.SemaphoreType.DMA for _ in range(3 * NBUF)],
      compiler_params=pltpu.CompilerParams(use_tc_tiling_on_sc=False),
  )
  def tok_pos_embed(idx_hbm, tok_hbm, pos_hbm, out_hbm, idx_all,
                    *bufs_and_sems):
    bufs = bufs_and_sems[:NBUF]
    ssem = bufs_and_sems[NBUF:2 * NBUF]
    gsem = bufs_and_sems[2 * NBUF:3 * NBUF]
    osem = bufs_and_sems[3 * NBUF:]
    wid = lax.axis_index("s") * NC + lax.axis_index("c")
    row0 = wid * ROWS_PER_W

    pltpu.sync_copy(idx_hbm.at[pl.ds(row0, ROWS_PER_W)], idx_all)

    seeds, gathers, outs = {}, {}, {}
    for r in range(ROWS_PER_W + 2):
      a = r
      if a < ROWS_PER_W:
        p = a % NBUF
        if a >= NBUF:
          outs.pop(a - NBUF).wait()
        seeds[a] = pltpu.async_copy(pos_hbm, bufs[p], ssem[p])
      b = r - 1
      if 0 <= b < ROWS_PER_W:
        p = b % NBUF
        seeds.pop(b).wait()
        gathers[b] = [
            pltpu.async_copy(
                tok_hbm.at[idx_all.at[b, j]],
                bufs[p].at[pl.ds(j * IDX_CHUNK, IDX_CHUNK)],
                gsem[p],
                add=True,
            )
            for j in range(IDX_SPLIT)
        ]
      c = r - 2
      if 0 <= c < ROWS_PER_W:
        p = c % NBUF
        for d in gathers.pop(c):
          d.wait()
        outs[c] = pltpu.async_copy(bufs[p], out_hbm.at[row0 + c], osem[p])

    for d in outs.values():
      d.wait()

  return tok_pos_embed


_KERNEL = _make_kernel()


def kernel(inputs, token_table, pos_table):
  idx = inputs.astype(jnp.int32).reshape(BATCH, IDX_SPLIT, IDX_CHUNK)
  return _KERNEL(idx, token_table, pos_table)


# speedup vs baseline: 1.0001x; 1.0001x over previous
import functools

import jax
import jax.numpy as jnp
from jax import lax
from jax.experimental import pallas as pl
from jax.experimental.pallas import tpu as pltpu
from jax.experimental.pallas import tpu_sc as plsc

VOCAB = 100000
DIM = 64
MAXLEN = 200
BATCH = 1024

NC = 2
NS = 16
NW = NC * NS
ROWS_PER_W = BATCH // NW

IDX_SPLIT = 2
IDX_CHUNK = MAXLEN // IDX_SPLIT

NBUF = 3


def _make_kernel():
  mesh = plsc.VectorSubcoreMesh(core_axis_name="c", subcore_axis_name="s")

  @functools.partial(
      pl.kernel,
      out_type=jax.ShapeDtypeStruct((BATCH, MAXLEN, DIM), jnp.float32),
      mesh=mesh,
      scratch_types=[
          pltpu.VMEM((ROWS_PER_W, IDX_SPLIT, IDX_CHUNK), jnp.int32),
      ]
      + [pltpu.VMEM((MAXLEN, DIM), jnp.float32) for _ in range(NBUF)]
      + [pltpu.SemaphoreType.DMA for _ in range(3 * NBUF)],
      compiler_params=pltpu.CompilerParams(use_tc_tiling_on_sc=False),
  )
  def tok_pos_embed(idx_hbm, tok_hbm, pos_hbm, out_hbm, idx_all,
                    *bufs_and_sems):
    bufs = bufs_and_sems[:NBUF]
    ssem = bufs_and_sems[NBUF:2 * NBUF]
    gsem = bufs_and_sems[2 * NBUF:3 * NBUF]
    osem = bufs_and_sems[3 * NBUF:]
    wid = lax.axis_index("s") * NC + lax.axis_index("c")
    row0 = wid * ROWS_PER_W

    pltpu.sync_copy(idx_hbm.at[pl.ds(row0, ROWS_PER_W)], idx_all)

    seeds, gathers, outs = {}, {}, {}
    for r in range(ROWS_PER_W + 2):
      a = r
      if a < ROWS_PER_W:
        p = a % NBUF
        if a >= NBUF:
          outs.pop(a - NBUF).wait()
        seeds[a] = pltpu.async_copy(pos_hbm, bufs[p], ssem[p])
      b = r - 1
      if 0 <= b < ROWS_PER_W:
        p = b % NBUF
        seeds.pop(b).wait()
        gathers[b] = [
            pltpu.async_copy(
                tok_hbm.at[idx_all.at[b, j]],
                bufs[p].at[pl.ds(j * IDX_CHUNK, IDX_CHUNK)],
                gsem[p],
                add=True,
            )
            for j in range(IDX_SPLIT)
        ]
      c = r - 2
      if 0 <= c < ROWS_PER_W:
        p = c % NBUF
        for d in gathers.pop(c):
          d.wait()
        outs[c] = pltpu.async_copy(bufs[p], out_hbm.at[row0 + c], osem[p])

    for d in outs.values():
      d.wait()

  return tok_pos_embed


_KERNEL = _make_kernel()


def kernel(inputs, token_table, pos_table):
  idx = inputs.astype(jnp.int32).reshape(BATCH, IDX_SPLIT, IDX_CHUNK)
  return _KERNEL(idx, token_table, pos_table)


# speedup vs baseline: 1.2739x; 1.2738x over previous
import functools

import jax
import jax.numpy as jnp
from jax import lax
from jax.experimental import pallas as pl
from jax.experimental.pallas import tpu as pltpu
from jax.experimental.pallas import tpu_sc as plsc

VOCAB = 100000
DIM = 64
PDIM = 128
MAXLEN = 200
BATCH = 1024

NC = 2
NS = 16
NW = NC * NS
ROWS_PER_W = BATCH // NW

IDX_SPLIT = 2
IDX_CHUNK = MAXLEN // IDX_SPLIT


def _make_kernel():
  mesh = plsc.VectorSubcoreMesh(core_axis_name="c", subcore_axis_name="s")

  @functools.partial(
      pl.kernel,
      out_type=jax.ShapeDtypeStruct((BATCH, MAXLEN, PDIM), jnp.float32),
      mesh=mesh,
      scratch_types=[
          pltpu.VMEM((ROWS_PER_W, IDX_SPLIT, IDX_CHUNK), jnp.int32),
          pltpu.VMEM((MAXLEN, DIM), jnp.float32),
          pltpu.SemaphoreType.DMA,
      ],
      compiler_params=pltpu.CompilerParams(use_tc_tiling_on_sc=False),
  )
  def tok_pos_embed(idx_hbm, tok_hbm, pos_hbm, out_hbm, idx_all, buf, sem):
    wid = lax.axis_index("s") * NC + lax.axis_index("c")
    row0 = wid * ROWS_PER_W

    pltpu.sync_copy(idx_hbm.at[pl.ds(row0, ROWS_PER_W)], idx_all)

    def body(r, carry):
      pltpu.sync_copy(pos_hbm, buf)
      cps = [
          pltpu.async_copy(
              tok_hbm.at[idx_all.at[r, j]],
              buf.at[pl.ds(j * IDX_CHUNK, IDX_CHUNK)],
              sem,
              add=True,
          )
          for j in range(IDX_SPLIT)
      ]
      for cp in cps:
        cp.wait()
      pltpu.sync_copy(buf, out_hbm.at[row0 + r].at[:, pl.ds(0, DIM)])
      return carry

    lax.fori_loop(0, ROWS_PER_W, body, 0)

  return tok_pos_embed


_KERNEL = _make_kernel()


def kernel(inputs, token_table, pos_table):
  idx = inputs.astype(jnp.int32).reshape(BATCH, IDX_SPLIT, IDX_CHUNK)
  out = _KERNEL(idx, token_table, pos_table)
  return out[:, :, :DIM]
